# baseline (speedup 1.0000x reference)
import jax
import jax.numpy as jnp
from jax import lax
from jax.experimental import pallas as pl
from jax.experimental.pallas import tpu as pltpu
from jax.experimental.pallas import tpu_sc as plsc

VOCAB = 1000000
EMB = 64
MAX_SEQ = 200
B = 1024
S = 200

NC = 2
NS = 16
NW = NC * NS

N_TOK = B * S
TPW = N_TOK // NW
SEQPW = TPW // S
NCH = SEQPW
NPAIR = NCH // 2
GROUPS = S // 16
TAIL = S - 16 * GROUPS
IDX_SLICES = (128, 72)

_RSQRT_MAGIC = 0x5F3759DF


def _rsqrt(v):
    i = plsc.bitcast(v, jnp.int32)
    y = plsc.bitcast(_RSQRT_MAGIC - (i >> 1), jnp.float32)
    half_v = 0.5 * v
    for _ in range(3):
        y = y * (1.5 - half_v * y * y)
    return y


def _sc_kernel(ids, pids, word, pos, lnw, lnb, out,
               idx_all, pid_all, pos_v, wb_v,
               rows0, rows1, out0, out1,
               gsem0, gsem1, osem0, osem1):
    wid = lax.axis_index("c") * NS + lax.axis_index("s")
    tok0 = pl.multiple_of(wid * TPW, TPW)
    seq0 = pl.multiple_of(wid * SEQPW, SEQPW)

    pltpu.sync_copy(ids.at[pl.ds(tok0, TPW)], idx_all)
    pltpu.sync_copy(pids.at[pl.ds(tok0, TPW)], pid_all)
    pltpu.sync_copy(pos, pos_v)
    pltpu.sync_copy(lnw, wb_v.at[0])
    pltpu.sync_copy(lnb, wb_v.at[1])

    w_q = [wb_v[0, pl.ds(16 * q, 16)] for q in range(EMB // 16)]
    b_q = [wb_v[1, pl.ds(16 * q, 16)] for q in range(EMB // 16)]
    iota16 = lax.iota(jnp.int32, 16)
    bfly = [iota16 ^ s for s in (8, 4, 2, 1)]
    dnums = lax.GatherDimensionNumbers(
        offset_dims=(), collapsed_slice_dims=(0,), start_index_map=(0,))

    def allsum(v):
        for p in bfly:
            perm = lax.gather(
                v, p[:, None], dimension_numbers=dnums, slice_sizes=(1,),
                mode=lax.GatherScatterMode.PROMISE_IN_BOUNDS)
            v = v + perm
        return v

    def fire_gather(ch, rows_v, sem):
        off = 0
        for n in IDX_SLICES:
            pltpu.make_async_copy(
                word.at[idx_all.at[pl.ds(ch * S + off, n)]],
                rows_v.at[pl.ds(off, n)],
                sem,
            ).start()
            off += n

    def wait_gather(rows_v, sem):
        off = 0
        for n in IDX_SLICES:
            pltpu.make_async_copy(
                word.at[idx_all.at[pl.ds(0, n)]],
                rows_v.at[pl.ds(off, n)],
                sem,
            ).wait()
            off += n

    def fire_out(ch, out_v, sem):
        pltpu.make_async_copy(out_v, out.at[seq0 + ch], sem).start()

    def wait_out(out_v, sem):
        pltpu.make_async_copy(out_v, out.at[seq0], sem).wait()

    def compute(ch, rows_v, out_v):
        def do_row(r, pid, k, out_v):
            pk = lax.squeeze(lax.slice(pid, (k,), (k + 1,)), (0,))
            x_q = []
            for q in range(EMB // 16):
                wv = rows_v[r, pl.ds(16 * q, 16)]
                pv = pos_v[pk, pl.ds(16 * q, 16)]
                x_q.append(wv + pv)
            t = (x_q[0] + x_q[1]) + (x_q[2] + x_q[3])
            t2 = ((x_q[0] * x_q[0] + x_q[1] * x_q[1])
                  + (x_q[2] * x_q[2] + x_q[3] * x_q[3]))
            mean_v = allsum(t) * (1.0 / EMB)
            var_v = allsum(t2) * (1.0 / EMB) - mean_v * mean_v
            rstd_v = _rsqrt(var_v + 1e-12)
            for q in range(EMB // 16):
                a = w_q[q] * rstd_v
                c = b_q[q] - mean_v * a
                out_v[r, pl.ds(16 * q, 16)] = x_q[q] * a + c

        def do_group(g, _):
            base = g * 16
            pid = pid_all[pl.ds(ch * S + base, 16)]
            for k in range(16):
                do_row(base + k, pid, k, out_v)
            return ()

        lax.fori_loop(0, GROUPS, do_group, ())
        pid = pid_all[pl.ds(ch * S + S - 16, 16)]
        for k in range(TAIL):
            do_row(16 * GROUPS + k, pid, 16 - TAIL + k, out_v)

    fire_gather(0, rows0, gsem0)

    def do_pair(p, _):
        ch0 = 2 * p
        fire_gather(ch0 + 1, rows1, gsem1)
        wait_gather(rows0, gsem0)

        @pl.when(p > 0)
        def _():
            wait_out(out0, osem0)

        compute(ch0, rows0, out0)
        fire_out(ch0, out0, osem0)

        @pl.when(p < NPAIR - 1)
        def _():
            fire_gather(ch0 + 2, rows0, gsem0)

        wait_gather(rows1, gsem1)

        @pl.when(p > 0)
        def _():
            wait_out(out1, osem1)

        compute(ch0 + 1, rows1, out1)
        fire_out(ch0 + 1, out1, osem1)
        return ()

    lax.fori_loop(0, NPAIR, do_pair, ())
    wait_out(out0, osem0)
    wait_out(out1, osem1)


@jax.jit
def _run(ids, pids, word, pos, lnw, lnb):
    mesh = plsc.VectorSubcoreMesh(core_axis_name="c", subcore_axis_name="s")
    return pl.kernel(
        _sc_kernel,
        out_type=jax.ShapeDtypeStruct((B, S, EMB), jnp.float32),
        mesh=mesh,
        scratch_types=[
            pltpu.VMEM((TPW,), jnp.int32),
            pltpu.VMEM((TPW,), jnp.int32),
            pltpu.VMEM((MAX_SEQ + 1, EMB), jnp.float32),
            pltpu.VMEM((2, EMB), jnp.float32),
            pltpu.VMEM((S, 128), jnp.float32),
            pltpu.VMEM((S, 128), jnp.float32),
            pltpu.VMEM((S, EMB), jnp.float32),
            pltpu.VMEM((S, EMB), jnp.float32),
            pltpu.SemaphoreType.DMA,
            pltpu.SemaphoreType.DMA,
            pltpu.SemaphoreType.DMA,
            pltpu.SemaphoreType.DMA,
        ],
        compiler_params=pltpu.CompilerParams(
            use_tc_tiling_on_sc=False, needs_layout_passes=False),
    )(ids, pids, word, pos, lnw, lnb)


def kernel(input_ids, mask, word_table, pos_table, ln_weight, ln_bias):
    ids = input_ids.reshape(N_TOK)
    pids = (jnp.arange(1, 1 + S, dtype=jnp.int32)[None, :]
            * mask.astype(jnp.int32)).reshape(N_TOK)
    word2 = jnp.pad(word_table, ((0, 0), (0, 128 - EMB)))
    return _run(ids, pids, word2, pos_table, ln_weight, ln_bias)


# speedup vs baseline: 1.0046x; 1.0046x over previous
import jax
import jax.numpy as jnp
from jax import lax
from jax.experimental import pallas as pl
from jax.experimental.pallas import tpu as pltpu
from jax.experimental.pallas import tpu_sc as plsc

VOCAB = 1000000
EMB = 64
MAX_SEQ = 200
B = 1024
S = 200

NC = 2
NS = 16
NW = NC * NS

N_TOK = B * S
TPW = N_TOK // NW
CHUNK = 128
NCH = TPW // CHUNK
NPAIR = NCH // 2
GROUPS = CHUNK // 16
OROWS = CHUNK * EMB // 128

_RSQRT_MAGIC = 0x5F3759DF


def _rsqrt(v):
    i = plsc.bitcast(v, jnp.int32)
    y = plsc.bitcast(_RSQRT_MAGIC - (i >> 1), jnp.float32)
    half_v = 0.5 * v
    for _ in range(3):
        y = y * (1.5 - half_v * y * y)
    return y


def _sc_kernel(ids, pids, word, pos, lnw, lnb, out,
               idx_all, pid_all, pos_v, wb_v,
               rows0, rows1, out0, out1,
               gsem0, gsem1, osem0, osem1):
    wid = lax.axis_index("c") * NS + lax.axis_index("s")
    tok0 = pl.multiple_of(wid * TPW, TPW)
    orow0 = pl.multiple_of(wid * (TPW * EMB // 128), TPW * EMB // 128)

    pltpu.sync_copy(ids.at[pl.ds(tok0, TPW)], idx_all)
    pltpu.sync_copy(pids.at[pl.ds(tok0, TPW)], pid_all)
    pltpu.sync_copy(pos, pos_v)
    pltpu.sync_copy(lnw, wb_v.at[0])
    pltpu.sync_copy(lnb, wb_v.at[1])

    w_q = [wb_v[0, pl.ds(16 * q, 16)] for q in range(EMB // 16)]
    b_q = [wb_v[1, pl.ds(16 * q, 16)] for q in range(EMB // 16)]
    iota16 = lax.iota(jnp.int32, 16)
    bfly = [iota16 ^ s for s in (8, 4, 2, 1)]
    dnums = lax.GatherDimensionNumbers(
        offset_dims=(), collapsed_slice_dims=(0,), start_index_map=(0,))

    def allsum(v):
        for p in bfly:
            perm = lax.gather(
                v, p[:, None], dimension_numbers=dnums, slice_sizes=(1,),
                mode=lax.GatherScatterMode.PROMISE_IN_BOUNDS)
            v = v + perm
        return v

    def fire_gather(ch, rows_v, sem):
        pltpu.make_async_copy(
            word.at[idx_all.at[pl.ds(ch * CHUNK, CHUNK)]],
            rows_v,
            sem,
        ).start()

    def wait_gather(rows_v, sem):
        pltpu.make_async_copy(
            word.at[idx_all.at[pl.ds(0, CHUNK)]],
            rows_v,
            sem,
        ).wait()

    def fire_out(ch, out_v, sem):
        row = pl.multiple_of(orow0 + ch * OROWS, OROWS)
        pltpu.make_async_copy(
            out_v,
            out.at[pl.ds(row, OROWS)],
            sem,
        ).start()

    def wait_out(out_v, sem):
        pltpu.make_async_copy(
            out_v,
            out.at[pl.ds(orow0, OROWS)],
            sem,
        ).wait()

    def compute(ch, rows_v, out_v):
        def do_group(g, _):
            base = g * 16
            pid = pid_all[pl.ds(ch * CHUNK + base, 16)]
            for k in range(16):
                r = base + k
                pk = lax.squeeze(lax.slice(pid, (k,), (k + 1,)), (0,))
                x_q = []
                for q in range(EMB // 16):
                    wv = rows_v[r, pl.ds(16 * q, 16)]
                    pv = pos_v[pk, pl.ds(16 * q, 16)]
                    x_q.append(wv + pv)
                t = (x_q[0] + x_q[1]) + (x_q[2] + x_q[3])
                t2 = ((x_q[0] * x_q[0] + x_q[1] * x_q[1])
                      + (x_q[2] * x_q[2] + x_q[3] * x_q[3]))
                mean_v = allsum(t) * (1.0 / EMB)
                var_v = allsum(t2) * (1.0 / EMB) - mean_v * mean_v
                rstd_v = _rsqrt(var_v + 1e-12)
                for q in range(EMB // 16):
                    a = w_q[q] * rstd_v
                    c = b_q[q] - mean_v * a
                    orow = g * 8 + (4 * k + q) // 8
                    ocol = 16 * ((4 * k + q) % 8)
                    out_v[orow, pl.ds(ocol, 16)] = x_q[q] * a + c
            return ()

        lax.fori_loop(0, GROUPS, do_group, ())

    fire_gather(0, rows0, gsem0)

    def do_pair(p, _):
        ch0 = 2 * p
        fire_gather(ch0 + 1, rows1, gsem1)
        wait_gather(rows0, gsem0)

        @pl.when(p > 0)
        def _():
            wait_out(out0, osem0)

        compute(ch0, rows0, out0)
        fire_out(ch0, out0, osem0)

        @pl.when(p < NPAIR - 1)
        def _():
            fire_gather(ch0 + 2, rows0, gsem0)

        wait_gather(rows1, gsem1)

        @pl.when(p > 0)
        def _():
            wait_out(out1, osem1)

        compute(ch0 + 1, rows1, out1)
        fire_out(ch0 + 1, out1, osem1)
        return ()

    lax.fori_loop(0, NPAIR, do_pair, ())
    wait_out(out0, osem0)
    wait_out(out1, osem1)


@jax.jit
def _run(ids, pids, word, pos, lnw, lnb):
    mesh = plsc.VectorSubcoreMesh(core_axis_name="c", subcore_axis_name="s")
    return pl.kernel(
        _sc_kernel,
        out_type=jax.ShapeDtypeStruct((N_TOK * EMB // 128, 128), jnp.float32),
        mesh=mesh,
        scratch_types=[
            pltpu.VMEM((TPW,), jnp.int32),
            pltpu.VMEM((TPW,), jnp.int32),
            pltpu.VMEM((MAX_SEQ + 1, EMB), jnp.float32),
            pltpu.VMEM((2, EMB), jnp.float32),
            pltpu.VMEM((CHUNK, 128), jnp.float32),
            pltpu.VMEM((CHUNK, 128), jnp.float32),
            pltpu.VMEM((OROWS, 128), jnp.float32),
            pltpu.VMEM((OROWS, 128), jnp.float32),
            pltpu.SemaphoreType.DMA,
            pltpu.SemaphoreType.DMA,
            pltpu.SemaphoreType.DMA,
            pltpu.SemaphoreType.DMA,
        ],
        compiler_params=pltpu.CompilerParams(
            use_tc_tiling_on_sc=False, needs_layout_passes=False),
    )(ids, pids, word, pos, lnw, lnb)


def kernel(input_ids, mask, word_table, pos_table, ln_weight, ln_bias):
    ids = input_ids.reshape(N_TOK)
    pids = (jnp.arange(1, 1 + S, dtype=jnp.int32)[None, :]
            * mask.astype(jnp.int32)).reshape(N_TOK)
    word2 = jnp.pad(word_table, ((0, 0), (0, 128 - EMB)))
    out = _run(ids, pids, word2, pos_table, ln_weight, ln_bias)
    return out.reshape(B, S, EMB)
